# baseline (speedup 1.0000x reference)
import functools

import jax
import jax.numpy as jnp
from jax import lax
from jax.experimental import pallas as pl
from jax.experimental.pallas import tpu as pltpu
from jax.experimental.pallas import tpu_sc as plsc

EMBED_DIM = 128
BATCH = 16384

_info = plsc.get_sparse_core_info()
_NC, _NS = _info.num_cores, _info.num_subcores
_NW = _NC * _NS
_B_PER_W = BATCH // _NW

_mesh = plsc.VectorSubcoreMesh(core_axis_name="c", subcore_axis_name="s")

_CHUNK = 128
_NCHUNK = _B_PER_W // _CHUNK


@functools.partial(
    pl.kernel,
    mesh=_mesh,
    out_type=jax.ShapeDtypeStruct((BATCH, EMBED_DIM), jnp.float32),
    scratch_types=[
        pltpu.VMEM((_NCHUNK, _CHUNK), jnp.int32),
        [pltpu.VMEM((_CHUNK, EMBED_DIM), jnp.float32) for _ in range(_NCHUNK)],
        pltpu.SemaphoreType.DMA,
        pltpu.SemaphoreType.DMA,
    ],
)
def _gather_kernel(ts_hbm, pe_hbm, out_hbm, idx_v, bufs, gsem, wsem):
    wid = lax.axis_index("s") * _NC + lax.axis_index("c")
    base = wid * _B_PER_W
    idx_copies = [
        pltpu.async_copy(
            ts_hbm.at[pl.ds(base + i * _CHUNK, _CHUNK)], idx_v.at[i], wsem
        )
        for i in range(_NCHUNK)
    ]
    for c in idx_copies:
        c.wait()
    gathers = [
        pltpu.async_copy(pe_hbm.at[idx_v.at[i]], bufs[i], gsem)
        for i in range(_NCHUNK)
    ]
    writes = []
    for i in range(_NCHUNK):
        gathers[i].wait()
        writes.append(
            pltpu.async_copy(
                bufs[i], out_hbm.at[pl.ds(base + i * _CHUNK, _CHUNK)], wsem
            )
        )
    for w in writes:
        w.wait()


def kernel(timesteps, pe):
    return _gather_kernel(timesteps.astype(jnp.int32), pe)


# speedup vs baseline: 1.1190x; 1.1190x over previous
import functools

import jax
import jax.numpy as jnp
from jax import lax
from jax.experimental import pallas as pl
from jax.experimental.pallas import tpu as pltpu
from jax.experimental.pallas import tpu_sc as plsc

EMBED_DIM = 128
MAX_TIMESTEPS = 1000
BATCH = 16384

_info = plsc.get_sparse_core_info()
_NC, _NS = _info.num_cores, _info.num_subcores
_NW = _NC * _NS
_B_PER_W = BATCH // _NW

_mesh = plsc.VectorSubcoreMesh(core_axis_name="c", subcore_axis_name="s")


@functools.partial(
    pl.kernel,
    mesh=_mesh,
    out_type=jax.ShapeDtypeStruct((BATCH, EMBED_DIM), jnp.float32),
    scratch_types=[
        pltpu.VMEM_SHARED((MAX_TIMESTEPS, EMBED_DIM), jnp.float32),
        pltpu.VMEM((_B_PER_W,), jnp.int32),
        pltpu.VMEM((_B_PER_W, EMBED_DIM), jnp.float32),
        pltpu.SemaphoreType.DMA,
    ],
)
def _gather_kernel(ts_hbm, pe_hbm, out_hbm, pe_sh, idx_v, rows_v, sem):
    cid = lax.axis_index("c")
    sid = lax.axis_index("s")
    wid = sid * _NC + cid
    base = wid * _B_PER_W

    @pl.when(sid < 15)
    def _stage():
        pltpu.sync_copy(
            pe_hbm.at[pl.ds(sid * 64, 64)], pe_sh.at[pl.ds(sid * 64, 64)]
        )

    @pl.when(sid == 15)
    def _stage_tail():
        pltpu.sync_copy(pe_hbm.at[pl.ds(960, 40)], pe_sh.at[pl.ds(960, 40)])

    pltpu.sync_copy(ts_hbm.at[pl.ds(base, _B_PER_W)], idx_v)
    plsc.subcore_barrier()
    pltpu.async_copy(pe_sh.at[idx_v], rows_v, sem).wait()
    pltpu.sync_copy(rows_v, out_hbm.at[pl.ds(base, _B_PER_W)])


def kernel(timesteps, pe):
    return _gather_kernel(timesteps.astype(jnp.int32), pe)


# speedup vs baseline: 1.1534x; 1.0307x over previous
import functools

import jax
import jax.numpy as jnp
from jax import lax
from jax.experimental import pallas as pl
from jax.experimental.pallas import tpu as pltpu
from jax.experimental.pallas import tpu_sc as plsc

EMBED_DIM = 128
MAX_TIMESTEPS = 1000
BATCH = 16384

_info = plsc.get_sparse_core_info()
_NC, _NS = _info.num_cores, _info.num_subcores
_NW = _NC * _NS
_B_PER_W = BATCH // _NW

_mesh = plsc.VectorSubcoreMesh(core_axis_name="c", subcore_axis_name="s")

_CHUNK = 128
_NCHUNK = _B_PER_W // _CHUNK


@functools.partial(
    pl.kernel,
    mesh=_mesh,
    out_type=jax.ShapeDtypeStruct((BATCH, EMBED_DIM), jnp.float32),
    scratch_types=[
        pltpu.VMEM_SHARED((MAX_TIMESTEPS, EMBED_DIM), jnp.float32),
        pltpu.VMEM((_NCHUNK, _CHUNK), jnp.int32),
        [pltpu.VMEM((_CHUNK, EMBED_DIM), jnp.float32) for _ in range(_NCHUNK)],
        pltpu.SemaphoreType.DMA,
        pltpu.SemaphoreType.DMA,
    ],
)
def _gather_kernel(ts_hbm, pe_hbm, out_hbm, pe_sh, idx_v, bufs, gsem, wsem):
    cid = lax.axis_index("c")
    sid = lax.axis_index("s")
    wid = sid * _NC + cid
    base = wid * _B_PER_W

    @pl.when(sid < 15)
    def _stage():
        pltpu.sync_copy(
            pe_hbm.at[pl.ds(sid * 64, 64)], pe_sh.at[pl.ds(sid * 64, 64)]
        )

    @pl.when(sid == 15)
    def _stage_tail():
        pltpu.sync_copy(pe_hbm.at[pl.ds(960, 40)], pe_sh.at[pl.ds(960, 40)])

    idx_copies = [
        pltpu.async_copy(
            ts_hbm.at[pl.ds(base + i * _CHUNK, _CHUNK)], idx_v.at[i], wsem
        )
        for i in range(_NCHUNK)
    ]
    for c in idx_copies:
        c.wait()
    plsc.subcore_barrier()
    gathers = [
        pltpu.async_copy(pe_sh.at[idx_v.at[i]], bufs[i], gsem)
        for i in range(_NCHUNK)
    ]
    writes = []
    for i in range(_NCHUNK):
        gathers[i].wait()
        writes.append(
            pltpu.async_copy(
                bufs[i], out_hbm.at[pl.ds(base + i * _CHUNK, _CHUNK)], wsem
            )
        )
    for w in writes:
        w.wait()


def kernel(timesteps, pe):
    return _gather_kernel(timesteps.astype(jnp.int32), pe)
